# baseline (speedup 1.0000x reference)
import functools

import jax
import jax.numpy as jnp
from jax import lax
from jax.experimental import pallas as pl
from jax.experimental.pallas import tpu as pltpu
from jax.experimental.pallas import tpu_sc as plsc

BATCH = 16384
EMB = 32
NC = 2
NS = 16
NW = NC * NS
BPW = BATCH // NW

_mesh = plsc.VectorSubcoreMesh(core_axis_name="c", subcore_axis_name="s")
_params = pltpu.CompilerParams(
    needs_layout_passes=False, use_tc_tiling_on_sc=False)


@functools.partial(
    pl.kernel,
    out_type=jax.ShapeDtypeStruct((NC, BATCH), jnp.float32),
    mesh=_mesh,
    scratch_types=[
        pltpu.VMEM((BATCH,), jnp.int32),
        pltpu.VMEM((BATCH,), jnp.int32),
        pltpu.VMEM((BATCH,), jnp.int32),
        pltpu.VMEM((BATCH,), jnp.float32),
        pltpu.VMEM((BATCH,), jnp.float32),
        pltpu.VMEM_SHARED((BATCH,), jnp.float32),
        pltpu.SemaphoreType.DMA,
        pltpu.SemaphoreType.DMA,
    ],
    compiler_params=_params,
)
def _mf_planes(u_hbm, v_hbm, ue_t, ve_t, part_hbm,
               uidx, vidx, iidx, gu, gv, accum, sem_u, sem_i):
    core = lax.axis_index("c")
    sid = lax.axis_index("s")
    j = sid * NC + core

    pltpu.sync_copy(u_hbm, uidx)
    pltpu.sync_copy(v_hbm, vidx)
    cp_u = pltpu.async_copy(ue_t.at[j].at[uidx], gu, sem_u)
    cp_v = pltpu.async_copy(ve_t.at[j].at[vidx], gv, sem_i)

    lanes = lax.iota(jnp.int32, 16)

    def fill(k, carry):
        for t in range(8):
            o = (k * 8 + t) * 16
            iidx[pl.ds(o, 16)] = o + lanes
        return carry

    lax.fori_loop(0, BATCH // (16 * 8), fill, 0)

    cp_u.wait()
    cp_v.wait()

    def body(k, carry):
        for t in range(8):
            s = pl.ds((k * 8 + t) * 16, 16)
            gu[s] = gu[s] * gv[s]
        return carry

    lax.fori_loop(0, BATCH // (16 * 8), body, 0)

    @pl.when(sid == 0)
    def _():
        pltpu.sync_copy(gu, accum)

    plsc.subcore_barrier()

    @pl.when(sid != 0)
    def _():
        pltpu.sync_copy(gu, accum.at[iidx], add=True)

    plsc.subcore_barrier()

    @pl.when(sid == 0)
    def _():
        pltpu.sync_copy(accum, part_hbm.at[core])


@functools.partial(
    pl.kernel,
    out_type=jax.ShapeDtypeStruct((BATCH,), jnp.float32),
    mesh=_mesh,
    scratch_types=[
        pltpu.VMEM((BPW,), jnp.float32),
        pltpu.VMEM((BPW,), jnp.float32),
        pltpu.VMEM((BPW,), jnp.float32),
    ],
    compiler_params=_params,
)
def _combine(part_hbm, out_hbm, pa, pb, ob):
    wid = lax.axis_index("s") * NC + lax.axis_index("c")
    base = wid * BPW
    pltpu.sync_copy(part_hbm.at[0, pl.ds(base, BPW)], pa)
    pltpu.sync_copy(part_hbm.at[1, pl.ds(base, BPW)], pb)

    def body(k, carry):
        s = pl.ds(k * 16, 16)
        ob[s] = pa[s] + pb[s]
        return carry

    lax.fori_loop(0, BPW // 16, body, 0)
    pltpu.sync_copy(ob, out_hbm.at[pl.ds(base, BPW)])


def kernel(u, v, user_emb, item_emb):
    part = _mf_planes(u.astype(jnp.int32), v.astype(jnp.int32),
                      user_emb.T, item_emb.T)
    return _combine(part)


# speedup vs baseline: 19.2555x; 19.2555x over previous
import functools

import jax
import jax.numpy as jnp
from jax import lax
from jax.experimental import pallas as pl
from jax.experimental.pallas import tpu as pltpu
from jax.experimental.pallas import tpu_sc as plsc

BATCH = 16384
EMB = 32
NC = 2
NS = 16
NW = NC * NS
BPW = BATCH // NW

NROWS = 1000000
TCOLS = 7813
PLANE_WORDS = TCOLS * 1024
NTILES = 4 * TCOLS
FLAT = NTILES * 1024

_mesh = plsc.VectorSubcoreMesh(core_axis_name="c", subcore_axis_name="s")
_tiled_params = pltpu.CompilerParams(needs_layout_passes=False)
_untiled_params = pltpu.CompilerParams(
    needs_layout_passes=False, use_tc_tiling_on_sc=False)

CHUNK = 32
FULL_CHUNKS = 244
TAIL_TILES = 4


@functools.partial(
    pl.kernel,
    out_type=(jax.ShapeDtypeStruct((NTILES, 8, 128), jnp.float32),
              jax.ShapeDtypeStruct((NTILES, 8, 128), jnp.float32)),
    mesh=_mesh,
    scratch_types=[
        pltpu.VMEM((2, 8, CHUNK * 128), jnp.float32),
        [pltpu.SemaphoreType.DMA] * 2,
        [pltpu.SemaphoreType.DMA] * 2,
    ],
    compiler_params=_tiled_params,
)
def _detile(ue_t, ve_t, uf_hbm, vf_hbm, bufs, sem_r, sem_w):
    wid = lax.axis_index("s") * NC + lax.axis_index("c")
    table = wid % 2
    a = (wid // 2) % 4
    q = wid // 8
    row0 = pl.multiple_of(8 * a, 8)
    c0 = q * 62
    n = jnp.where(q == 3, FULL_CHUNKS - 3 * 62, 62)

    def do_table(src, dst):
        def read(c, slot):
            col = pl.multiple_of((c0 + c) * CHUNK * 128, 128)
            pltpu.async_copy(
                src.at[pl.ds(row0, 8), pl.ds(col, CHUNK * 128)],
                bufs.at[slot], sem_r[slot])

        def wait_read(slot):
            pltpu.make_async_copy(
                src.at[pl.ds(0, 8), pl.ds(0, CHUNK * 128)],
                bufs.at[slot], sem_r[slot]).wait()

        def write_chunk(c, slot):
            t0 = a * TCOLS + (c0 + c) * CHUNK
            for grp in range(4):
                cps = []
                for k in range(8):
                    t = grp * 8 + k
                    cps.append(pltpu.async_copy(
                        bufs.at[slot, :, pl.ds(t * 128, 128)],
                        dst.at[t0 + t], sem_w[slot]))
                for cp in cps:
                    cp.wait()

        read(0, 0)
        read(1, 1)

        def body(g, carry):
            for slot in range(2):
                c = g * 2 + slot
                wait_read(slot)
                write_chunk(c, slot)

                @pl.when(c + 2 < n)
                def _():
                    read(c + 2, slot)
            return carry

        lax.fori_loop(0, n // 2, body, 0)

        @pl.when(q == 3)
        def _():
            colt = pl.multiple_of(FULL_CHUNKS * CHUNK * 128, 128)
            width = TAIL_TILES * 128
            pltpu.async_copy(
                src.at[pl.ds(row0, 8), pl.ds(colt, width)],
                bufs.at[0, :, pl.ds(0, width)], sem_r[0]).wait()
            t0 = a * TCOLS + FULL_CHUNKS * CHUNK
            cps = []
            for k in range(TAIL_TILES):
                cps.append(pltpu.async_copy(
                    bufs.at[0, :, pl.ds(k * 128, 128)],
                    dst.at[t0 + k], sem_w[0]))
            for cp in cps:
                cp.wait()

    @pl.when(table == 0)
    def _():
        do_table(ue_t, uf_hbm)

    @pl.when(table == 1)
    def _():
        do_table(ve_t, vf_hbm)


@functools.partial(
    pl.kernel,
    out_type=jax.ShapeDtypeStruct((NC, BATCH), jnp.float32),
    mesh=_mesh,
    scratch_types=[
        pltpu.VMEM((BATCH,), jnp.int32),
        pltpu.VMEM((BATCH,), jnp.int32),
        pltpu.VMEM((BATCH,), jnp.int32),
        pltpu.VMEM((BATCH,), jnp.float32),
        pltpu.VMEM((BATCH,), jnp.float32),
        pltpu.VMEM((64, EMB), jnp.float32),
        pltpu.VMEM((64, EMB), jnp.float32),
        pltpu.VMEM_SHARED((BATCH,), jnp.float32),
        pltpu.SemaphoreType.DMA,
        pltpu.SemaphoreType.DMA,
    ],
    compiler_params=_untiled_params,
)
def _mf_planes(u_hbm, v_hbm, uflat, vflat, utail, vtail, part_hbm,
               uidx, vidx, iidx, gu, gv, ut_v, vt_v, accum, sem_u, sem_i):
    core = lax.axis_index("c")
    sid = lax.axis_index("s")
    j = sid * NC + core
    base = (j // 8) * PLANE_WORDS + (j % 8) * 128

    pltpu.sync_copy(u_hbm, uidx)
    pltpu.sync_copy(v_hbm, vidx)
    pltpu.sync_copy(utail, ut_v)
    pltpu.sync_copy(vtail, vt_v)

    lanes = lax.iota(jnp.int32, 16)

    def to_words(k, carry):
        for t in range(8):
            s = pl.ds((k * 8 + t) * 16, 16)
            w = uidx[s]
            uidx[s] = base + lax.shift_left(
                lax.shift_right_logical(w, 7), 10) + (w & 127)
            w = vidx[s]
            vidx[s] = base + lax.shift_left(
                lax.shift_right_logical(w, 7), 10) + (w & 127)
        return carry

    lax.fori_loop(0, BATCH // (16 * 8), to_words, 0)

    cp_u = pltpu.async_copy(uflat.at[uidx], gu, sem_u)
    cp_v = pltpu.async_copy(vflat.at[vidx], gv, sem_i)

    def fill(k, carry):
        for t in range(8):
            o = (k * 8 + t) * 16
            iidx[pl.ds(o, 16)] = o + lanes
        return carry

    lax.fori_loop(0, BATCH // (16 * 8), fill, 0)

    cp_u.wait()
    cp_v.wait()

    jvec = jnp.broadcast_to(j, (16,)).astype(jnp.int32)
    tail0 = 7812 * 1024

    def body(k, carry):
        for t in range(8):
            s = pl.ds((k * 8 + t) * 16, 16)
            rel_u = uidx[s] - base
            rel_v = vidx[s] - base
            tu = jnp.clip(rel_u - tail0, 0, 63)
            tv = jnp.clip(rel_v - tail0, 0, 63)
            au = plsc.load_gather(ut_v, [tu, jvec])
            av = plsc.load_gather(vt_v, [tv, jvec])
            pu = jnp.where(rel_u >= tail0, au, gu[s])
            pv = jnp.where(rel_v >= tail0, av, gv[s])
            gu[s] = pu * pv
        return carry

    lax.fori_loop(0, BATCH // (16 * 8), body, 0)

    @pl.when(sid == 0)
    def _():
        pltpu.sync_copy(gu, accum)

    plsc.subcore_barrier()

    @pl.when(sid != 0)
    def _():
        pltpu.sync_copy(gu, accum.at[iidx], add=True)

    plsc.subcore_barrier()

    @pl.when(sid == 0)
    def _():
        pltpu.sync_copy(accum, part_hbm.at[core])


@functools.partial(
    pl.kernel,
    out_type=jax.ShapeDtypeStruct((BATCH,), jnp.float32),
    mesh=_mesh,
    scratch_types=[
        pltpu.VMEM((BPW,), jnp.float32),
        pltpu.VMEM((BPW,), jnp.float32),
        pltpu.VMEM((BPW,), jnp.float32),
    ],
    compiler_params=_untiled_params,
)
def _combine(part_hbm, out_hbm, pa, pb, ob):
    wid = lax.axis_index("s") * NC + lax.axis_index("c")
    base = wid * BPW
    pltpu.sync_copy(part_hbm.at[0, pl.ds(base, BPW)], pa)
    pltpu.sync_copy(part_hbm.at[1, pl.ds(base, BPW)], pb)

    def body(k, carry):
        s = pl.ds(k * 16, 16)
        ob[s] = pa[s] + pb[s]
        return carry

    lax.fori_loop(0, BPW // 16, body, 0)
    pltpu.sync_copy(ob, out_hbm.at[pl.ds(base, BPW)])


def kernel(u, v, user_emb, item_emb):
    uf3, vf3 = _detile(user_emb.T, item_emb.T)
    uflat = uf3.reshape(FLAT)
    vflat = vf3.reshape(FLAT)
    part = _mf_planes(u.astype(jnp.int32), v.astype(jnp.int32),
                      uflat, vflat,
                      user_emb[7812 * 128:], item_emb[7812 * 128:])
    return _combine(part)


# speedup vs baseline: 19.4702x; 1.0111x over previous
import functools

import jax
import jax.numpy as jnp
from jax import lax
from jax.experimental import pallas as pl
from jax.experimental.pallas import tpu as pltpu
from jax.experimental.pallas import tpu_sc as plsc

BATCH = 16384
EMB = 32
NC = 2
NS = 16
NW = NC * NS
BPW = BATCH // NW

NROWS = 1000000
TCOLS = 7813
PLANE_WORDS = TCOLS * 1024
NTILES = 4 * TCOLS
FLAT = NTILES * 1024

_mesh = plsc.VectorSubcoreMesh(core_axis_name="c", subcore_axis_name="s")
_tiled_params = pltpu.CompilerParams(needs_layout_passes=False)
_untiled_params = pltpu.CompilerParams(
    needs_layout_passes=False, use_tc_tiling_on_sc=False)

CHUNK = 32
FULL_CHUNKS = 244
TAIL_TILES = 4


@functools.partial(
    pl.kernel,
    out_type=(jax.ShapeDtypeStruct((NTILES, 8, 128), jnp.float32),
              jax.ShapeDtypeStruct((NTILES, 8, 128), jnp.float32)),
    mesh=_mesh,
    scratch_types=[
        pltpu.VMEM((2, 8, CHUNK * 128), jnp.float32),
        [pltpu.SemaphoreType.DMA] * 2,
        [pltpu.SemaphoreType.DMA] * 2,
    ],
    compiler_params=_tiled_params,
)
def _detile(ue_t, ve_t, uf_hbm, vf_hbm, bufs, sem_r, sem_w):
    wid = lax.axis_index("s") * NC + lax.axis_index("c")
    table = wid % 2
    a = (wid // 2) % 4
    q = wid // 8
    row0 = pl.multiple_of(8 * a, 8)
    c0 = q * 62
    n = jnp.where(q == 3, FULL_CHUNKS - 3 * 62, 62)

    def do_table(src, dst):
        def read(c, slot):
            col = pl.multiple_of((c0 + c) * CHUNK * 128, 128)
            pltpu.async_copy(
                src.at[pl.ds(row0, 8), pl.ds(col, CHUNK * 128)],
                bufs.at[slot], sem_r[slot])

        def wait_read(slot):
            pltpu.make_async_copy(
                src.at[pl.ds(0, 8), pl.ds(0, CHUNK * 128)],
                bufs.at[slot], sem_r[slot]).wait()

        def write_chunk(c, slot):
            t0 = a * TCOLS + (c0 + c) * CHUNK
            for grp in range(4):
                cps = []
                for k in range(8):
                    t = grp * 8 + k
                    cps.append(pltpu.async_copy(
                        bufs.at[slot, :, pl.ds(t * 128, 128)],
                        dst.at[t0 + t], sem_w[slot]))
                for cp in cps:
                    cp.wait()

        read(0, 0)
        read(1, 1)

        def body(g, carry):
            for slot in range(2):
                c = g * 2 + slot
                wait_read(slot)
                write_chunk(c, slot)

                @pl.when(c + 2 < n)
                def _():
                    read(c + 2, slot)
            return carry

        lax.fori_loop(0, n // 2, body, 0)

        @pl.when(q == 3)
        def _():
            colt = pl.multiple_of(FULL_CHUNKS * CHUNK * 128, 128)
            width = TAIL_TILES * 128
            pltpu.async_copy(
                src.at[pl.ds(row0, 8), pl.ds(colt, width)],
                bufs.at[0, :, pl.ds(0, width)], sem_r[0]).wait()
            t0 = a * TCOLS + FULL_CHUNKS * CHUNK
            cps = []
            for k in range(TAIL_TILES):
                cps.append(pltpu.async_copy(
                    bufs.at[0, :, pl.ds(k * 128, 128)],
                    dst.at[t0 + k], sem_w[0]))
            for cp in cps:
                cp.wait()

    @pl.when(table == 0)
    def _():
        do_table(ue_t, uf_hbm)

    @pl.when(table == 1)
    def _():
        do_table(ve_t, vf_hbm)


@functools.partial(
    pl.kernel,
    out_type=jax.ShapeDtypeStruct((NC, BATCH), jnp.float32),
    mesh=_mesh,
    scratch_types=[
        pltpu.VMEM((BATCH,), jnp.int32),
        pltpu.VMEM((BATCH,), jnp.int32),
        pltpu.VMEM((BATCH,), jnp.int32),
        pltpu.VMEM((BATCH,), jnp.float32),
        pltpu.VMEM((BATCH,), jnp.float32),
        pltpu.VMEM((64, EMB), jnp.float32),
        pltpu.VMEM((64, EMB), jnp.float32),
        pltpu.VMEM_SHARED((BATCH,), jnp.float32),
        [pltpu.SemaphoreType.DMA] * 4,
        [pltpu.SemaphoreType.DMA] * 4,
    ],
    compiler_params=_untiled_params,
)
def _mf_planes(u_hbm, v_hbm, uflat, vflat, utail, vtail, part_hbm,
               uidx, vidx, iidx, gu, gv, ut_v, vt_v, accum, sem_u, sem_i):
    core = lax.axis_index("c")
    sid = lax.axis_index("s")
    j = sid * NC + core
    base = (j // 8) * PLANE_WORDS + (j % 8) * 128
    CH = BATCH // 4

    pltpu.sync_copy(u_hbm, uidx)
    pltpu.sync_copy(v_hbm, vidx)
    pltpu.sync_copy(utail, ut_v)
    pltpu.sync_copy(vtail, vt_v)

    lanes = lax.iota(jnp.int32, 16)

    def to_words(idxref, c):
        def step(k, carry):
            for t in range(8):
                s = pl.ds(c * CH + (k * 8 + t) * 16, 16)
                w = idxref[s]
                idxref[s] = base + lax.shift_left(
                    lax.shift_right_logical(w, 7), 10) + (w & 127)
            return carry

        lax.fori_loop(0, CH // (16 * 8), step, 0)

    cps = []
    for c in range(4):
        cs = pl.ds(c * CH, CH)
        to_words(uidx, c)
        cps.append(pltpu.async_copy(
            uflat.at[uidx.at[cs]], gu.at[cs], sem_u[c]))
        to_words(vidx, c)
        cps.append(pltpu.async_copy(
            vflat.at[vidx.at[cs]], gv.at[cs], sem_i[c]))

    def fill(k, carry):
        for t in range(8):
            o = (k * 8 + t) * 16
            iidx[pl.ds(o, 16)] = o + lanes
        return carry

    lax.fori_loop(0, BATCH // (16 * 8), fill, 0)

    jvec = jnp.broadcast_to(j, (16,)).astype(jnp.int32)
    tail0 = 7812 * 1024

    def product(c):
        def step(k, carry):
            for t in range(8):
                s = pl.ds(c * CH + (k * 8 + t) * 16, 16)
                rel_u = uidx[s] - base
                rel_v = vidx[s] - base
                tu = jnp.clip(rel_u - tail0, 0, 63)
                tv = jnp.clip(rel_v - tail0, 0, 63)
                au = plsc.load_gather(ut_v, [tu, jvec])
                av = plsc.load_gather(vt_v, [tv, jvec])
                pu = jnp.where(rel_u >= tail0, au, gu[s])
                pv = jnp.where(rel_v >= tail0, av, gv[s])
                gu[s] = pu * pv
            return carry

        lax.fori_loop(0, CH // (16 * 8), step, 0)

    for c in range(4):
        cps[2 * c].wait()
        cps[2 * c + 1].wait()
        product(c)

    @pl.when(sid == 0)
    def _():
        pltpu.sync_copy(gu, accum)

    plsc.subcore_barrier()

    @pl.when(sid != 0)
    def _():
        pltpu.sync_copy(gu, accum.at[iidx], add=True)

    plsc.subcore_barrier()

    @pl.when(sid == 0)
    def _():
        pltpu.sync_copy(accum, part_hbm.at[core])


@functools.partial(
    pl.kernel,
    out_type=jax.ShapeDtypeStruct((BATCH,), jnp.float32),
    mesh=_mesh,
    scratch_types=[
        pltpu.VMEM((BPW,), jnp.float32),
        pltpu.VMEM((BPW,), jnp.float32),
        pltpu.VMEM((BPW,), jnp.float32),
    ],
    compiler_params=_untiled_params,
)
def _combine(part_hbm, out_hbm, pa, pb, ob):
    wid = lax.axis_index("s") * NC + lax.axis_index("c")
    base = wid * BPW
    pltpu.sync_copy(part_hbm.at[0, pl.ds(base, BPW)], pa)
    pltpu.sync_copy(part_hbm.at[1, pl.ds(base, BPW)], pb)

    def body(k, carry):
        s = pl.ds(k * 16, 16)
        ob[s] = pa[s] + pb[s]
        return carry

    lax.fori_loop(0, BPW // 16, body, 0)
    pltpu.sync_copy(ob, out_hbm.at[pl.ds(base, BPW)])


def kernel(u, v, user_emb, item_emb):
    uf3, vf3 = _detile(user_emb.T, item_emb.T)
    uflat = uf3.reshape(FLAT)
    vflat = vf3.reshape(FLAT)
    part = _mf_planes(u.astype(jnp.int32), v.astype(jnp.int32),
                      uflat, vflat,
                      user_emb[7812 * 128:], item_emb[7812 * 128:])
    return _combine(part)
